# Initial kernel scaffold; baseline (speedup 1.0000x reference)
import functools

import jax
import jax.numpy as jnp
from jax.experimental import pallas as pl
from jax.experimental.pallas import tpu as pltpu

_NPOINTS = [4096, 1024, 256, 64]
_RADIUS = [[0.1, 0.5], [0.5, 1.0], [1.0, 2.0], [2.0, 4.0]]
_NSAMPLE = [[16, 32], [16, 32], [16, 32], [16, 32]]


def _rup(x, m):
    return (x + m - 1) // m * m


def _sa_mlp_body(g_ref, nx_ref, *wrefs, o_ref, ns, nlayers):
    eye_ref = wrefs[-1]
    wr = wrefs[:-1]
    x = g_ref[...]
    q = nx_ref.shape[0]
    nz = nx_ref[...]
    nz3 = jnp.broadcast_to(nz[:, None, :], (q, ns, 3)).reshape(q * ns, 3)
    nzfull = jax.lax.dot_general(
        nz3, eye_ref[...], (((1,), (0,)), ((), ())),
        preferred_element_type=jnp.float32)
    h = x - nzfull
    for i in range(nlayers):
        w = wr[2 * i][...]
        b = wr[2 * i + 1][...]
        h = jax.lax.dot_general(h, w, (((1,), (0,)), ((), ())),
                                preferred_element_type=jnp.float32)
        h = jnp.maximum(h + b, 0.0)
    c3 = h.shape[-1]
    o_ref[...] = jnp.max(h.reshape(q, ns, c3), axis=1)


def _sa_mlp(g, nxyz, layers, ns):
    rows, dpad0 = g.shape
    r = nxyz.shape[0]
    nlayers = len(layers)
    q = min(r, max(1, 2048 // ns))
    while r % q:
        q //= 2
    wargs = []
    wspecs = []
    dpad = dpad0
    for (w, b) in layers:
        wt = jnp.zeros((dpad, w.shape[0]), jnp.float32).at[: w.shape[1], :].set(w.T)
        dpad = w.shape[0]
        wargs += [wt, b[None, :]]
        wspecs += [
            pl.BlockSpec(wt.shape, lambda i: (0, 0)),
            pl.BlockSpec((1, b.shape[0]), lambda i: (0, 0)),
        ]
    eye = jnp.zeros((3, dpad0), jnp.float32).at[:, :3].set(jnp.eye(3))
    wargs.append(eye)
    wspecs.append(pl.BlockSpec(eye.shape, lambda i: (0, 0)))
    c3 = layers[-1][0].shape[0]
    out = pl.pallas_call(
        functools.partial(_sa_mlp_body, ns=ns, nlayers=nlayers),
        grid=(r // q,),
        in_specs=[
            pl.BlockSpec((q * ns, dpad0), lambda i: (i, 0)),
            pl.BlockSpec((q, 3), lambda i: (i, 0)),
        ] + wspecs,
        out_specs=pl.BlockSpec((q, c3), lambda i: (i, 0)),
        out_shape=jax.ShapeDtypeStruct((r, c3), jnp.float32),
    )(g, nxyz, *wargs)
    return out


def _fp_body(ux_ref, kx_ref, uf_ref, kf_ref, *wrefs, o_ref, nlayers):
    u = ux_ref.shape[1]
    nk = kx_ref.shape[1]
    ux = ux_ref[0]
    kx = kx_ref[0]
    u2 = (ux[:, 0] * ux[:, 0] + ux[:, 1] * ux[:, 1] + ux[:, 2] * ux[:, 2])[:, None]
    k2 = (kx[:, 0] * kx[:, 0] + kx[:, 1] * kx[:, 1] + kx[:, 2] * kx[:, 2])[None, :]
    dot = (ux[:, 0:1] * kx[:, 0][None, :]
           + ux[:, 1:2] * kx[:, 1][None, :]
           + ux[:, 2:3] * kx[:, 2][None, :])
    d2 = u2 + k2 - 2.0 * dot
    iota = jax.lax.broadcasted_iota(jnp.int32, (u, nk), 1)
    d2w = d2
    ms, ps = [], []
    for _ in range(3):
        mj = jnp.min(d2w, axis=1, keepdims=True)
        pj = jnp.min(jnp.where(d2w == mj, iota, nk), axis=1, keepdims=True)
        d2w = jnp.where(iota == pj, jnp.float32(1e30), d2w)
        ms.append(mj)
        ps.append(pj)
    rs = [1.0 / (jnp.maximum(m, 0.0) + 1e-8) for m in ms]
    wsum = (rs[0] + rs[1]) + rs[2]
    mmat = ((rs[0] / wsum) * (iota == ps[0]).astype(jnp.float32)
            + (rs[1] / wsum) * (iota == ps[1]).astype(jnp.float32)
            + (rs[2] / wsum) * (iota == ps[2]).astype(jnp.float32))
    interp = jax.lax.dot_general(mmat, kf_ref[0], (((1,), (0,)), ((), ())),
                                 preferred_element_type=jnp.float32)
    uf = uf_ref[0]
    h = None
    for i in range(nlayers):
        if i == 0:
            wa = wrefs[0][...]
            wb = wrefs[1][...]
            b = wrefs[2][...]
            h = (jax.lax.dot_general(interp, wa, (((1,), (0,)), ((), ())),
                                     preferred_element_type=jnp.float32)
                 + jax.lax.dot_general(uf, wb, (((1,), (0,)), ((), ())),
                                       preferred_element_type=jnp.float32))
            h = jnp.maximum(h + b, 0.0)
        else:
            w = wrefs[3 * i][...]
            b = wrefs[3 * i + 1][...]
            h = jax.lax.dot_general(h, w, (((1,), (0,)), ((), ())),
                                    preferred_element_type=jnp.float32)
            h = jnp.maximum(h + b, 0.0)
    o_ref[0] = h


def _fp(uxyz, kxyz, ufeat, kfeat, layers):
    bsz, nu, _ = uxyz.shape
    nk = kxyz.shape[1]
    cu = ufeat.shape[2]
    ck = kfeat.shape[2]
    nlayers = len(layers)
    u = min(nu, 512)
    wargs, wspecs = [], []
    for i, (w, b) in enumerate(layers):
        if i == 0:
            wa = w[:, :ck].T
            wb = w[:, ck:].T
            wargs += [wa, wb, b[None, :]]
            wspecs += [pl.BlockSpec(wa.shape, lambda bi, i: (0, 0)),
                       pl.BlockSpec(wb.shape, lambda bi, i: (0, 0)),
                       pl.BlockSpec((1, b.shape[0]), lambda bi, i: (0, 0))]
        else:
            wt = w.T
            wargs += [wt, b[None, :], b[None, :1]]
            wspecs += [pl.BlockSpec(wt.shape, lambda bi, i: (0, 0)),
                       pl.BlockSpec((1, b.shape[0]), lambda bi, i: (0, 0)),
                       pl.BlockSpec((1, 1), lambda bi, i: (0, 0))]
    cout = layers[-1][0].shape[0]
    out = pl.pallas_call(
        functools.partial(_fp_body, nlayers=nlayers),
        grid=(bsz, nu // u),
        in_specs=[
            pl.BlockSpec((1, u, 3), lambda bi, i: (bi, i, 0)),
            pl.BlockSpec((1, nk, 3), lambda bi, i: (bi, 0, 0)),
            pl.BlockSpec((1, u, cu), lambda bi, i: (bi, i, 0)),
            pl.BlockSpec((1, nk, ck), lambda bi, i: (bi, 0, 0)),
        ] + wspecs,
        out_specs=pl.BlockSpec((1, u, cout), lambda bi, i: (bi, i, 0)),
        out_shape=jax.ShapeDtypeStruct((bsz, nu, cout), jnp.float32),
    )(uxyz, kxyz, ufeat, kfeat, *wargs)
    return out


def _fps_jax(xyz, npoint):
    bn, nn, _ = xyz.shape
    dists = jnp.full((bn, nn), 1e10, dtype=jnp.float32)
    idxs = jnp.zeros((bn, npoint), dtype=jnp.int32)

    def body(i, state):
        dists, idxs = state
        last = idxs[:, i - 1]
        last_pt = jnp.take_along_axis(xyz, last[:, None, None], axis=1)
        d = jnp.sum((xyz - last_pt) ** 2, axis=-1)
        dists = jnp.minimum(dists, d)
        nxt = jnp.argmax(dists, axis=1).astype(jnp.int32)
        idxs = idxs.at[:, i].set(nxt)
        return (dists, idxs)

    dists, idxs = jax.lax.fori_loop(1, npoint, body, (dists, idxs))
    return idxs


def _sqdist(a, b):
    a2 = jnp.sum(a * a, axis=-1)
    b2 = jnp.sum(b * b, axis=-1)
    return a2[:, :, None] + b2[:, None, :] - 2.0 * jnp.einsum('bnd,bmd->bnm', a, b)


def _ball_query_jax(radius, nsample, xyz, new_xyz):
    nn = xyz.shape[1]
    d2 = _sqdist(new_xyz, xyz)
    mask = d2 < radius * radius
    ar = jnp.arange(nn, dtype=jnp.int32)
    keys = jnp.where(mask, -ar[None, None, :], jnp.int32(-(nn + 1)))
    vals, _ = jax.lax.top_k(keys, nsample)
    idx = -vals
    first = jnp.where(idx[:, :, :1] >= nn, 0, idx[:, :, :1])
    idx = jnp.where(idx >= nn, first, idx)
    return idx


def _sa_level(xyz, featrows, k, sa_k):
    bsz, n, _ = xyz.shape
    npoint = _NPOINTS[k]
    fid = _fps_jax(jax.lax.stop_gradient(xyz), npoint)
    new_xyz = jnp.take_along_axis(xyz, fid[:, :, None], axis=1)
    nxyz_rows = new_xyz.reshape(bsz * npoint, 3)
    table = jnp.concatenate([xyz, featrows], axis=-1)
    d = table.shape[-1]
    dpad = _rup(d, 16)
    if dpad != d:
        table = jnp.concatenate(
            [table, jnp.zeros((bsz, n, dpad - d), jnp.float32)], axis=-1)
    outs = []
    for s in range(2):
        r, ns = _RADIUS[k][s], _NSAMPLE[k][s]
        idx = _ball_query_jax(r, ns, xyz, new_xyz)
        g = jnp.take_along_axis(table[:, None, :, :], idx[:, :, :, None], axis=2)
        g = g.reshape(bsz * npoint * ns, dpad)
        o = _sa_mlp(g, nxyz_rows, sa_k[s], ns)
        outs.append(o.reshape(bsz, npoint, -1))
    return new_xyz, jnp.concatenate(outs, axis=-1)


def kernel(pointcloud, sa_params, fp_params):
    xyz = pointcloud[:, :, 0:3]
    featrows = pointcloud[:, :, 3:]
    l_xyz = [xyz]
    l_feat = [featrows]
    for k in range(4):
        nx, nf = _sa_level(l_xyz[k], l_feat[k], k, sa_params[k])
        l_xyz.append(nx)
        l_feat.append(nf)
    for i in range(-1, -5, -1):
        l_feat[i - 1] = _fp(l_xyz[i - 1], l_xyz[i], l_feat[i - 1], l_feat[i],
                            fp_params[i])
    return l_feat[0]


# speedup vs baseline: 1.2874x; 1.2874x over previous
import functools

import jax
import jax.numpy as jnp
from jax.experimental import pallas as pl
from jax.experimental.pallas import tpu as pltpu

_NPOINTS = [4096, 1024, 256, 64]
_RADIUS = [[0.1, 0.5], [0.5, 1.0], [1.0, 2.0], [2.0, 4.0]]
_NSAMPLE = [[16, 32], [16, 32], [16, 32], [16, 32]]


def _rup(x, m):
    return (x + m - 1) // m * m


def _sa_mlp_body(g_ref, nx_ref, *refs, ns, nlayers):
    o_ref = refs[-1]
    wr = refs[:-1]
    x = g_ref[...]
    q = nx_ref.shape[0]
    dpad = x.shape[1]
    nz = nx_ref[...]
    nz3 = jnp.broadcast_to(nz[:, None, :], (q, ns, 3)).reshape(q * ns, 3)
    colv = jax.lax.broadcasted_iota(jnp.int32, (q * ns, dpad), 1)
    nzfull = (jnp.where(colv == 0, nz3[:, 0:1], 0.0)
              + jnp.where(colv == 1, nz3[:, 1:2], 0.0)
              + jnp.where(colv == 2, nz3[:, 2:3], 0.0))
    h = x - nzfull
    for i in range(nlayers):
        w = wr[2 * i][...]
        b = wr[2 * i + 1][...]
        h = jax.lax.dot_general(h, w, (((1,), (0,)), ((), ())),
                                preferred_element_type=jnp.float32)
        h = jnp.maximum(h + b, 0.0)
    c3 = h.shape[-1]
    o_ref[...] = jnp.max(h.reshape(q, ns, c3), axis=1)


def _sa_mlp(g, nxyz, layers, ns):
    rows, dpad0 = g.shape
    r = nxyz.shape[0]
    nlayers = len(layers)
    q = min(r, max(1, 2048 // ns))
    while r % q:
        q //= 2
    wargs = []
    wspecs = []
    dpad = dpad0
    for (w, b) in layers:
        wt = jnp.zeros((dpad, w.shape[0]), jnp.float32).at[: w.shape[1], :].set(w.T)
        dpad = w.shape[0]
        wargs += [wt, b[None, :]]
        wspecs += [
            pl.BlockSpec(wt.shape, lambda i: (0, 0)),
            pl.BlockSpec((1, b.shape[0]), lambda i: (0, 0)),
        ]
    c3 = layers[-1][0].shape[0]
    out = pl.pallas_call(
        functools.partial(_sa_mlp_body, ns=ns, nlayers=nlayers),
        grid=(r // q,),
        in_specs=[
            pl.BlockSpec((q * ns, dpad0), lambda i: (i, 0)),
            pl.BlockSpec((q, 3), lambda i: (i, 0)),
        ] + wspecs,
        out_specs=pl.BlockSpec((q, c3), lambda i: (i, 0)),
        out_shape=jax.ShapeDtypeStruct((r, c3), jnp.float32),
    )(g, nxyz, *wargs)
    return out


def _fp_body(ux_ref, kx_ref, uf_ref, kf_ref, *refs, nlayers):
    o_ref = refs[-1]
    wrefs = refs[:-1]
    u = ux_ref.shape[1]
    nk = kx_ref.shape[1]
    ux = ux_ref[0]
    kx = kx_ref[0]
    u2 = (ux[:, 0] * ux[:, 0] + ux[:, 1] * ux[:, 1] + ux[:, 2] * ux[:, 2])[:, None]
    k2 = (kx[:, 0] * kx[:, 0] + kx[:, 1] * kx[:, 1] + kx[:, 2] * kx[:, 2])[None, :]
    dot = jax.lax.dot_general(ux, kx, (((1,), (1,)), ((), ())),
                              preferred_element_type=jnp.float32)
    d2 = u2 + k2 - 2.0 * dot
    iota = jax.lax.broadcasted_iota(jnp.int32, (u, nk), 1)
    d2w = d2
    ms, ps = [], []
    for _ in range(3):
        mj = jnp.min(d2w, axis=1, keepdims=True)
        pj = jnp.min(jnp.where(d2w == mj, iota, nk), axis=1, keepdims=True)
        d2w = jnp.where(iota == pj, jnp.float32(1e30), d2w)
        ms.append(mj)
        ps.append(pj)
    rs = [1.0 / (jnp.maximum(m, 0.0) + 1e-8) for m in ms]
    wsum = (rs[0] + rs[1]) + rs[2]
    gs = []
    for j in range(3):
        onehot = (iota == ps[j]).astype(jnp.float32)
        gs.append(jax.lax.dot_general(
            onehot, kf_ref[0], (((1,), (0,)), ((), ())),
            precision=jax.lax.Precision.HIGHEST,
            preferred_element_type=jnp.float32))
    interp = (gs[0] * (rs[0] / wsum) + gs[1] * (rs[1] / wsum)
              + gs[2] * (rs[2] / wsum))
    uf = uf_ref[0]
    h = None
    for i in range(nlayers):
        if i == 0:
            wa = wrefs[0][...]
            wb = wrefs[1][...]
            b = wrefs[2][...]
            h = (jax.lax.dot_general(interp, wa, (((1,), (0,)), ((), ())),
                                     preferred_element_type=jnp.float32)
                 + jax.lax.dot_general(uf, wb, (((1,), (0,)), ((), ())),
                                       preferred_element_type=jnp.float32))
            h = jnp.maximum(h + b, 0.0)
        else:
            w = wrefs[3 * i][...]
            b = wrefs[3 * i + 1][...]
            h = jax.lax.dot_general(h, w, (((1,), (0,)), ((), ())),
                                    preferred_element_type=jnp.float32)
            h = jnp.maximum(h + b, 0.0)
    o_ref[0] = h


def _fp(uxyz, kxyz, ufeat, kfeat, layers):
    bsz, nu, _ = uxyz.shape
    nk = kxyz.shape[1]
    cu = ufeat.shape[2]
    ck = kfeat.shape[2]
    nlayers = len(layers)
    u = min(nu, 512)
    wargs, wspecs = [], []
    for i, (w, b) in enumerate(layers):
        if i == 0:
            wa = w[:, :ck].T
            wb = w[:, ck:].T
            wargs += [wa, wb, b[None, :]]
            wspecs += [pl.BlockSpec(wa.shape, lambda bi, i: (0, 0)),
                       pl.BlockSpec(wb.shape, lambda bi, i: (0, 0)),
                       pl.BlockSpec((1, b.shape[0]), lambda bi, i: (0, 0))]
        else:
            wt = w.T
            wargs += [wt, b[None, :], b[None, :1]]
            wspecs += [pl.BlockSpec(wt.shape, lambda bi, i: (0, 0)),
                       pl.BlockSpec((1, b.shape[0]), lambda bi, i: (0, 0)),
                       pl.BlockSpec((1, 1), lambda bi, i: (0, 0))]
    cout = layers[-1][0].shape[0]
    out = pl.pallas_call(
        functools.partial(_fp_body, nlayers=nlayers),
        grid=(bsz, nu // u),
        in_specs=[
            pl.BlockSpec((1, u, 3), lambda bi, i: (bi, i, 0)),
            pl.BlockSpec((1, nk, 3), lambda bi, i: (bi, 0, 0)),
            pl.BlockSpec((1, u, cu), lambda bi, i: (bi, i, 0)),
            pl.BlockSpec((1, nk, ck), lambda bi, i: (bi, 0, 0)),
        ] + wspecs,
        out_specs=pl.BlockSpec((1, u, cout), lambda bi, i: (bi, i, 0)),
        out_shape=jax.ShapeDtypeStruct((bsz, nu, cout), jnp.float32),
    )(uxyz, kxyz, ufeat, kfeat, *wargs)
    return out


def _fps_jax(xyz, npoint):
    bn, nn, _ = xyz.shape
    dists = jnp.full((bn, nn), 1e10, dtype=jnp.float32)
    idxs = jnp.zeros((bn, npoint), dtype=jnp.int32)

    def body(i, state):
        dists, idxs = state
        last = idxs[:, i - 1]
        last_pt = jnp.take_along_axis(xyz, last[:, None, None], axis=1)
        d = jnp.sum((xyz - last_pt) ** 2, axis=-1)
        dists = jnp.minimum(dists, d)
        nxt = jnp.argmax(dists, axis=1).astype(jnp.int32)
        idxs = idxs.at[:, i].set(nxt)
        return (dists, idxs)

    dists, idxs = jax.lax.fori_loop(1, npoint, body, (dists, idxs))
    return idxs


def _sqdist(a, b):
    a2 = jnp.sum(a * a, axis=-1)
    b2 = jnp.sum(b * b, axis=-1)
    return a2[:, :, None] + b2[:, None, :] - 2.0 * jnp.einsum('bnd,bmd->bnm', a, b)


def _ball_query_jax(radius, nsample, xyz, new_xyz):
    nn = xyz.shape[1]
    d2 = _sqdist(new_xyz, xyz)
    mask = d2 < radius * radius
    ar = jnp.arange(nn, dtype=jnp.int32)
    keys = jnp.where(mask, -ar[None, None, :], jnp.int32(-(nn + 1)))
    vals, _ = jax.lax.top_k(keys, nsample)
    idx = -vals
    first = jnp.where(idx[:, :, :1] >= nn, 0, idx[:, :, :1])
    idx = jnp.where(idx >= nn, first, idx)
    return idx


def _sa_level(xyz, featrows, k, sa_k):
    bsz, n, _ = xyz.shape
    npoint = _NPOINTS[k]
    fid = _fps_jax(jax.lax.stop_gradient(xyz), npoint)
    new_xyz = jnp.take_along_axis(xyz, fid[:, :, None], axis=1)
    nxyz_rows = new_xyz.reshape(bsz * npoint, 3)
    table = jnp.concatenate([xyz, featrows], axis=-1)
    d = table.shape[-1]
    dpad = _rup(d, 16)
    if dpad != d:
        table = jnp.concatenate(
            [table, jnp.zeros((bsz, n, dpad - d), jnp.float32)], axis=-1)
    outs = []
    for s in range(2):
        r, ns = _RADIUS[k][s], _NSAMPLE[k][s]
        idx = _ball_query_jax(r, ns, xyz, new_xyz)
        g = jnp.take_along_axis(table[:, None, :, :], idx[:, :, :, None], axis=2)
        g = g.reshape(bsz * npoint * ns, dpad)
        o = _sa_mlp(g, nxyz_rows, sa_k[s], ns)
        outs.append(o.reshape(bsz, npoint, -1))
    return new_xyz, jnp.concatenate(outs, axis=-1)


def kernel(pointcloud, sa_params, fp_params):
    xyz = pointcloud[:, :, 0:3]
    featrows = pointcloud[:, :, 3:]
    l_xyz = [xyz]
    l_feat = [featrows]
    for k in range(4):
        nx, nf = _sa_level(l_xyz[k], l_feat[k], k, sa_params[k])
        l_xyz.append(nx)
        l_feat.append(nf)
    for i in range(-1, -5, -1):
        l_feat[i - 1] = _fp(l_xyz[i - 1], l_xyz[i], l_feat[i - 1], l_feat[i],
                            fp_params[i])
    return l_feat[0]


# speedup vs baseline: 2.4944x; 1.9376x over previous
import functools

import jax
import jax.numpy as jnp
from jax.experimental import pallas as pl
from jax.experimental.pallas import tpu as pltpu

_NPOINTS = [4096, 1024, 256, 64]
_RADIUS = [[0.1, 0.5], [0.5, 1.0], [1.0, 2.0], [2.0, 4.0]]
_NSAMPLE = [[16, 32], [16, 32], [16, 32], [16, 32]]


def _rup(x, m):
    return (x + m - 1) // m * m


def _sa_mlp_body(g_ref, nx_ref, *refs, ns, nlayers):
    o_ref = refs[-1]
    wr = refs[:-1]
    x = g_ref[...]
    q = nx_ref.shape[0]
    dpad = x.shape[1]
    nz = nx_ref[...]
    nz3 = jnp.broadcast_to(nz[:, None, :], (q, ns, 3)).reshape(q * ns, 3)
    colv = jax.lax.broadcasted_iota(jnp.int32, (q * ns, dpad), 1)
    nzfull = (jnp.where(colv == 0, nz3[:, 0:1], 0.0)
              + jnp.where(colv == 1, nz3[:, 1:2], 0.0)
              + jnp.where(colv == 2, nz3[:, 2:3], 0.0))
    h = x - nzfull
    for i in range(nlayers):
        w = wr[2 * i][...]
        b = wr[2 * i + 1][...]
        h = jax.lax.dot_general(h, w, (((1,), (0,)), ((), ())),
                                preferred_element_type=jnp.float32)
        h = jnp.maximum(h + b, 0.0)
    c3 = h.shape[-1]
    o_ref[...] = jnp.max(h.reshape(q, ns, c3), axis=1)


def _sa_mlp(g, nxyz, layers, ns):
    rows, dpad0 = g.shape
    r = nxyz.shape[0]
    nlayers = len(layers)
    q = min(r, max(1, 2048 // ns))
    while r % q:
        q //= 2
    wargs = []
    wspecs = []
    dpad = dpad0
    for (w, b) in layers:
        wt = jnp.zeros((dpad, w.shape[0]), jnp.float32).at[: w.shape[1], :].set(w.T)
        dpad = w.shape[0]
        wargs += [wt, b[None, :]]
        wspecs += [
            pl.BlockSpec(wt.shape, lambda i: (0, 0)),
            pl.BlockSpec((1, b.shape[0]), lambda i: (0, 0)),
        ]
    c3 = layers[-1][0].shape[0]
    out = pl.pallas_call(
        functools.partial(_sa_mlp_body, ns=ns, nlayers=nlayers),
        grid=(r // q,),
        in_specs=[
            pl.BlockSpec((q * ns, dpad0), lambda i: (i, 0)),
            pl.BlockSpec((q, 3), lambda i: (i, 0)),
        ] + wspecs,
        out_specs=pl.BlockSpec((q, c3), lambda i: (i, 0)),
        out_shape=jax.ShapeDtypeStruct((r, c3), jnp.float32),
    )(g, nxyz, *wargs)
    return out


def _fp_body(ux_ref, kx_ref, uf_ref, kf_ref, *refs, nlayers):
    o_ref = refs[-1]
    wrefs = refs[:-1]
    u = ux_ref.shape[1]
    nk = kx_ref.shape[1]
    ux = ux_ref[0]
    kx = kx_ref[0]
    u2 = (ux[:, 0] * ux[:, 0] + ux[:, 1] * ux[:, 1] + ux[:, 2] * ux[:, 2])[:, None]
    k2 = (kx[:, 0] * kx[:, 0] + kx[:, 1] * kx[:, 1] + kx[:, 2] * kx[:, 2])[None, :]
    dot = jax.lax.dot_general(ux, kx, (((1,), (1,)), ((), ())),
                              preferred_element_type=jnp.float32)
    d2 = u2 + k2 - 2.0 * dot
    iota = jax.lax.broadcasted_iota(jnp.int32, (u, nk), 1)
    d2w = d2
    ms, ps = [], []
    for _ in range(3):
        mj = jnp.min(d2w, axis=1, keepdims=True)
        pj = jnp.min(jnp.where(d2w == mj, iota, nk), axis=1, keepdims=True)
        d2w = jnp.where(iota == pj, jnp.float32(1e30), d2w)
        ms.append(mj)
        ps.append(pj)
    rs = [1.0 / (jnp.maximum(m, 0.0) + 1e-8) for m in ms]
    wsum = (rs[0] + rs[1]) + rs[2]
    gs = []
    for j in range(3):
        onehot = (iota == ps[j]).astype(jnp.float32)
        gs.append(jax.lax.dot_general(
            onehot, kf_ref[0], (((1,), (0,)), ((), ())),
            precision=jax.lax.Precision.HIGHEST,
            preferred_element_type=jnp.float32))
    interp = (gs[0] * (rs[0] / wsum) + gs[1] * (rs[1] / wsum)
              + gs[2] * (rs[2] / wsum))
    uf = uf_ref[0]
    h = None
    for i in range(nlayers):
        if i == 0:
            wa = wrefs[0][...]
            wb = wrefs[1][...]
            b = wrefs[2][...]
            h = (jax.lax.dot_general(interp, wa, (((1,), (0,)), ((), ())),
                                     preferred_element_type=jnp.float32)
                 + jax.lax.dot_general(uf, wb, (((1,), (0,)), ((), ())),
                                       preferred_element_type=jnp.float32))
            h = jnp.maximum(h + b, 0.0)
        else:
            w = wrefs[3 * i][...]
            b = wrefs[3 * i + 1][...]
            h = jax.lax.dot_general(h, w, (((1,), (0,)), ((), ())),
                                    preferred_element_type=jnp.float32)
            h = jnp.maximum(h + b, 0.0)
    o_ref[0] = h


def _fp(uxyz, kxyz, ufeat, kfeat, layers):
    bsz, nu, _ = uxyz.shape
    nk = kxyz.shape[1]
    cu = ufeat.shape[2]
    ck = kfeat.shape[2]
    nlayers = len(layers)
    u = min(nu, 512)
    wargs, wspecs = [], []
    for i, (w, b) in enumerate(layers):
        if i == 0:
            wa = w[:, :ck].T
            wb = w[:, ck:].T
            wargs += [wa, wb, b[None, :]]
            wspecs += [pl.BlockSpec(wa.shape, lambda bi, i: (0, 0)),
                       pl.BlockSpec(wb.shape, lambda bi, i: (0, 0)),
                       pl.BlockSpec((1, b.shape[0]), lambda bi, i: (0, 0))]
        else:
            wt = w.T
            wargs += [wt, b[None, :], b[None, :1]]
            wspecs += [pl.BlockSpec(wt.shape, lambda bi, i: (0, 0)),
                       pl.BlockSpec((1, b.shape[0]), lambda bi, i: (0, 0)),
                       pl.BlockSpec((1, 1), lambda bi, i: (0, 0))]
    cout = layers[-1][0].shape[0]
    out = pl.pallas_call(
        functools.partial(_fp_body, nlayers=nlayers),
        grid=(bsz, nu // u),
        in_specs=[
            pl.BlockSpec((1, u, 3), lambda bi, i: (bi, i, 0)),
            pl.BlockSpec((1, nk, 3), lambda bi, i: (bi, 0, 0)),
            pl.BlockSpec((1, u, cu), lambda bi, i: (bi, i, 0)),
            pl.BlockSpec((1, nk, ck), lambda bi, i: (bi, 0, 0)),
        ] + wspecs,
        out_specs=pl.BlockSpec((1, u, cout), lambda bi, i: (bi, i, 0)),
        out_shape=jax.ShapeDtypeStruct((bsz, nu, cout), jnp.float32),
    )(uxyz, kxyz, ufeat, kfeat, *wargs)
    return out


def _fps_body(xs_ref, ys_ref, zs_ref, idx_ref, nx_ref, d_ref, *, npoint):
    bsz, nb, _ = xs_ref.shape
    xs, ys, zs = xs_ref[...], ys_ref[...], zs_ref[...]
    gi = (jax.lax.broadcasted_iota(jnp.int32, (bsz, nb, 128), 1) * 128
          + jax.lax.broadcasted_iota(jnp.int32, (bsz, nb, 128), 2))
    d_ref[...] = jnp.full((bsz, nb, 128), 1e10, jnp.float32)
    idx_ref[...] = jnp.zeros(idx_ref.shape, jnp.int32)
    iota_np = jax.lax.broadcasted_iota(jnp.int32, idx_ref.shape, 1)
    iota_nx = jax.lax.broadcasted_iota(jnp.int32, nx_ref.shape, 2)

    def coords_of(li):
        sel = gi == li
        lx = jnp.sum(jnp.where(sel, xs, 0.0), axis=(1, 2), keepdims=True)
        ly = jnp.sum(jnp.where(sel, ys, 0.0), axis=(1, 2), keepdims=True)
        lz = jnp.sum(jnp.where(sel, zs, 0.0), axis=(1, 2), keepdims=True)
        return lx, ly, lz

    def body(i, last):
        lx, ly, lz = coords_of(last)
        dx, dy, dz = xs - lx, ys - ly, zs - lz
        d = (dx * dx + dy * dy) + dz * dz
        nd = jnp.minimum(d_ref[...], d)
        d_ref[...] = nd
        m = jnp.max(nd, axis=(1, 2), keepdims=True)
        nxt = jnp.min(jnp.where(nd == m, gi, jnp.int32(2 ** 30)),
                      axis=(1, 2), keepdims=True)
        idx_ref[...] = jnp.where(iota_np == i, nxt[:, :, 0], idx_ref[...])
        lc = jnp.concatenate([lx, ly, lz], axis=1)
        nx_ref[...] = jnp.where(iota_nx == i - 1, lc, nx_ref[...])
        return nxt

    last = jax.lax.fori_loop(1, npoint, body,
                             jnp.zeros((bsz, 1, 1), jnp.int32))
    lx, ly, lz = coords_of(last)
    lc = jnp.concatenate([lx, ly, lz], axis=1)
    nx_ref[...] = jnp.where(iota_nx == npoint - 1, lc, nx_ref[...])


def _fps_pallas(xyz, npoint):
    bsz, n, _ = xyz.shape
    nb = n // 128
    xs = xyz[:, :, 0].reshape(bsz, nb, 128)
    ys = xyz[:, :, 1].reshape(bsz, nb, 128)
    zs = xyz[:, :, 2].reshape(bsz, nb, 128)
    idx, nxyz = pl.pallas_call(
        functools.partial(_fps_body, npoint=npoint),
        out_shape=(jax.ShapeDtypeStruct((bsz, npoint), jnp.int32),
                   jax.ShapeDtypeStruct((bsz, 3, npoint), jnp.float32)),
        scratch_shapes=[pltpu.VMEM((bsz, nb, 128), jnp.float32)],
    )(xs, ys, zs)
    return idx, jnp.transpose(nxyz, (0, 2, 1))


def _fps_jax(xyz, npoint):
    bn, nn, _ = xyz.shape
    dists = jnp.full((bn, nn), 1e10, dtype=jnp.float32)
    idxs = jnp.zeros((bn, npoint), dtype=jnp.int32)

    def body(i, state):
        dists, idxs = state
        last = idxs[:, i - 1]
        last_pt = jnp.take_along_axis(xyz, last[:, None, None], axis=1)
        d = jnp.sum((xyz - last_pt) ** 2, axis=-1)
        dists = jnp.minimum(dists, d)
        nxt = jnp.argmax(dists, axis=1).astype(jnp.int32)
        idxs = idxs.at[:, i].set(nxt)
        return (dists, idxs)

    dists, idxs = jax.lax.fori_loop(1, npoint, body, (dists, idxs))
    return idxs


def _sqdist(a, b):
    a2 = jnp.sum(a * a, axis=-1)
    b2 = jnp.sum(b * b, axis=-1)
    return a2[:, :, None] + b2[:, None, :] - 2.0 * jnp.einsum('bnd,bmd->bnm', a, b)


def _ball_query_jax(radius, nsample, xyz, new_xyz):
    nn = xyz.shape[1]
    d2 = _sqdist(new_xyz, xyz)
    mask = d2 < radius * radius
    ar = jnp.arange(nn, dtype=jnp.int32)
    keys = jnp.where(mask, -ar[None, None, :], jnp.int32(-(nn + 1)))
    vals, _ = jax.lax.top_k(keys, nsample)
    idx = -vals
    first = jnp.where(idx[:, :, :1] >= nn, 0, idx[:, :, :1])
    idx = jnp.where(idx >= nn, first, idx)
    return idx


def _sa_level(xyz, featrows, k, sa_k):
    bsz, n, _ = xyz.shape
    npoint = _NPOINTS[k]
    fid, new_xyz = _fps_pallas(xyz, npoint)
    nxyz_rows = new_xyz.reshape(bsz * npoint, 3)
    table = jnp.concatenate([xyz, featrows], axis=-1)
    d = table.shape[-1]
    dpad = _rup(d, 16)
    if dpad != d:
        table = jnp.concatenate(
            [table, jnp.zeros((bsz, n, dpad - d), jnp.float32)], axis=-1)
    outs = []
    for s in range(2):
        r, ns = _RADIUS[k][s], _NSAMPLE[k][s]
        idx = _ball_query_jax(r, ns, xyz, new_xyz)
        g = jnp.take_along_axis(table[:, None, :, :], idx[:, :, :, None], axis=2)
        g = g.reshape(bsz * npoint * ns, dpad)
        o = _sa_mlp(g, nxyz_rows, sa_k[s], ns)
        outs.append(o.reshape(bsz, npoint, -1))
    return new_xyz, jnp.concatenate(outs, axis=-1)


def kernel(pointcloud, sa_params, fp_params):
    xyz = pointcloud[:, :, 0:3]
    featrows = pointcloud[:, :, 3:]
    l_xyz = [xyz]
    l_feat = [featrows]
    for k in range(4):
        nx, nf = _sa_level(l_xyz[k], l_feat[k], k, sa_params[k])
        l_xyz.append(nx)
        l_feat.append(nf)
    for i in range(-1, -5, -1):
        l_feat[i - 1] = _fp(l_xyz[i - 1], l_xyz[i], l_feat[i - 1], l_feat[i],
                            fp_params[i])
    return l_feat[0]


# speedup vs baseline: 3.1347x; 1.2567x over previous
import functools

import jax
import jax.numpy as jnp
from jax.experimental import pallas as pl
from jax.experimental.pallas import tpu as pltpu
from jax.experimental.pallas import tpu_sc as plsc

_NPOINTS = [4096, 1024, 256, 64]
_RADIUS = [[0.1, 0.5], [0.5, 1.0], [1.0, 2.0], [2.0, 4.0]]
_NSAMPLE = [[16, 32], [16, 32], [16, 32], [16, 32]]


def _rup(x, m):
    return (x + m - 1) // m * m


def _sa_mlp_body(g_ref, nx_ref, *refs, ns, nlayers):
    o_ref = refs[-1]
    wr = refs[:-1]
    x = g_ref[...]
    q = nx_ref.shape[0]
    dpad = x.shape[1]
    nz = nx_ref[...]
    nz3 = jnp.broadcast_to(nz[:, None, :], (q, ns, 3)).reshape(q * ns, 3)
    colv = jax.lax.broadcasted_iota(jnp.int32, (q * ns, dpad), 1)
    nzfull = (jnp.where(colv == 0, nz3[:, 0:1], 0.0)
              + jnp.where(colv == 1, nz3[:, 1:2], 0.0)
              + jnp.where(colv == 2, nz3[:, 2:3], 0.0))
    h = x - nzfull
    for i in range(nlayers):
        w = wr[2 * i][...]
        b = wr[2 * i + 1][...]
        h = jax.lax.dot_general(h, w, (((1,), (0,)), ((), ())),
                                preferred_element_type=jnp.float32)
        h = jnp.maximum(h + b, 0.0)
    c3 = h.shape[-1]
    o_ref[...] = jnp.max(h.reshape(q, ns, c3), axis=1)


def _sa_mlp(g, nxyz, layers, ns):
    rows, dpad0 = g.shape
    r = nxyz.shape[0]
    nlayers = len(layers)
    q = min(r, max(1, 2048 // ns))
    while r % q:
        q //= 2
    wargs = []
    wspecs = []
    dpad = dpad0
    for (w, b) in layers:
        wt = jnp.zeros((dpad, w.shape[0]), jnp.float32).at[: w.shape[1], :].set(w.T)
        dpad = w.shape[0]
        wargs += [wt, b[None, :]]
        wspecs += [
            pl.BlockSpec(wt.shape, lambda i: (0, 0)),
            pl.BlockSpec((1, b.shape[0]), lambda i: (0, 0)),
        ]
    c3 = layers[-1][0].shape[0]
    out = pl.pallas_call(
        functools.partial(_sa_mlp_body, ns=ns, nlayers=nlayers),
        grid=(r // q,),
        in_specs=[
            pl.BlockSpec((q * ns, dpad0), lambda i: (i, 0)),
            pl.BlockSpec((q, 3), lambda i: (i, 0)),
        ] + wspecs,
        out_specs=pl.BlockSpec((q, c3), lambda i: (i, 0)),
        out_shape=jax.ShapeDtypeStruct((r, c3), jnp.float32),
    )(g, nxyz, *wargs)
    return out


def _fp_body(ux_ref, kx_ref, uf_ref, kf_ref, *refs, nlayers):
    o_ref = refs[-1]
    wrefs = refs[:-1]
    u = ux_ref.shape[1]
    nk = kx_ref.shape[1]
    ux = ux_ref[0]
    kx = kx_ref[0]
    u2 = (ux[:, 0] * ux[:, 0] + ux[:, 1] * ux[:, 1] + ux[:, 2] * ux[:, 2])[:, None]
    k2 = (kx[:, 0] * kx[:, 0] + kx[:, 1] * kx[:, 1] + kx[:, 2] * kx[:, 2])[None, :]
    dot = jax.lax.dot_general(ux, kx, (((1,), (1,)), ((), ())),
                              preferred_element_type=jnp.float32)
    d2 = u2 + k2 - 2.0 * dot
    iota = jax.lax.broadcasted_iota(jnp.int32, (u, nk), 1)
    d2w = d2
    ms, ps = [], []
    for _ in range(3):
        mj = jnp.min(d2w, axis=1, keepdims=True)
        pj = jnp.min(jnp.where(d2w == mj, iota, nk), axis=1, keepdims=True)
        d2w = jnp.where(iota == pj, jnp.float32(1e30), d2w)
        ms.append(mj)
        ps.append(pj)
    rs = [1.0 / (jnp.maximum(m, 0.0) + 1e-8) for m in ms]
    wsum = (rs[0] + rs[1]) + rs[2]
    gs = []
    for j in range(3):
        onehot = (iota == ps[j]).astype(jnp.float32)
        gs.append(jax.lax.dot_general(
            onehot, kf_ref[0], (((1,), (0,)), ((), ())),
            precision=jax.lax.Precision.HIGHEST,
            preferred_element_type=jnp.float32))
    interp = (gs[0] * (rs[0] / wsum) + gs[1] * (rs[1] / wsum)
              + gs[2] * (rs[2] / wsum))
    uf = uf_ref[0]
    h = None
    for i in range(nlayers):
        if i == 0:
            wa = wrefs[0][...]
            wb = wrefs[1][...]
            b = wrefs[2][...]
            h = (jax.lax.dot_general(interp, wa, (((1,), (0,)), ((), ())),
                                     preferred_element_type=jnp.float32)
                 + jax.lax.dot_general(uf, wb, (((1,), (0,)), ((), ())),
                                       preferred_element_type=jnp.float32))
            h = jnp.maximum(h + b, 0.0)
        else:
            w = wrefs[3 * i][...]
            b = wrefs[3 * i + 1][...]
            h = jax.lax.dot_general(h, w, (((1,), (0,)), ((), ())),
                                    preferred_element_type=jnp.float32)
            h = jnp.maximum(h + b, 0.0)
    o_ref[0] = h


def _fp(uxyz, kxyz, ufeat, kfeat, layers):
    bsz, nu, _ = uxyz.shape
    nk = kxyz.shape[1]
    cu = ufeat.shape[2]
    ck = kfeat.shape[2]
    nlayers = len(layers)
    u = min(nu, 512)
    wargs, wspecs = [], []
    for i, (w, b) in enumerate(layers):
        if i == 0:
            wa = w[:, :ck].T
            wb = w[:, ck:].T
            wargs += [wa, wb, b[None, :]]
            wspecs += [pl.BlockSpec(wa.shape, lambda bi, i: (0, 0)),
                       pl.BlockSpec(wb.shape, lambda bi, i: (0, 0)),
                       pl.BlockSpec((1, b.shape[0]), lambda bi, i: (0, 0))]
        else:
            wt = w.T
            wargs += [wt, b[None, :], b[None, :1]]
            wspecs += [pl.BlockSpec(wt.shape, lambda bi, i: (0, 0)),
                       pl.BlockSpec((1, b.shape[0]), lambda bi, i: (0, 0)),
                       pl.BlockSpec((1, 1), lambda bi, i: (0, 0))]
    cout = layers[-1][0].shape[0]
    out = pl.pallas_call(
        functools.partial(_fp_body, nlayers=nlayers),
        grid=(bsz, nu // u),
        in_specs=[
            pl.BlockSpec((1, u, 3), lambda bi, i: (bi, i, 0)),
            pl.BlockSpec((1, nk, 3), lambda bi, i: (bi, 0, 0)),
            pl.BlockSpec((1, u, cu), lambda bi, i: (bi, i, 0)),
            pl.BlockSpec((1, nk, ck), lambda bi, i: (bi, 0, 0)),
        ] + wspecs,
        out_specs=pl.BlockSpec((1, u, cout), lambda bi, i: (bi, i, 0)),
        out_shape=jax.ShapeDtypeStruct((bsz, nu, cout), jnp.float32),
    )(uxyz, kxyz, ufeat, kfeat, *wargs)
    return out


def _d2_body(xr_ref, kt_ref, nxt_ref, o_ref):
    xr = xr_ref[0]
    kt = kt_ref[0]
    nxt = nxt_ref[...]
    k2 = ((xr[:, 0] * xr[:, 0] + xr[:, 1] * xr[:, 1]) + xr[:, 2] * xr[:, 2])[:, None]
    q2 = ((nxt[0] * nxt[0] + nxt[1] * nxt[1]) + nxt[2] * nxt[2])[None, :]
    dot = jax.lax.dot_general(kt, nxt, (((0,), (0,)), ((), ())),
                              preferred_element_type=jnp.float32)
    o_ref[0] = (q2 + k2) - 2.0 * dot


def _d2t_small_body(xr_ref, kt_ref, nxt_ref, o_ref, *, npoint):
    bsz = xr_ref.shape[0]
    for bi in range(bsz):
        xr = xr_ref[bi]
        kt = kt_ref[bi]
        nxt = nxt_ref[:, bi * npoint:(bi + 1) * npoint]
        k2 = ((xr[:, 0] * xr[:, 0] + xr[:, 1] * xr[:, 1])
              + xr[:, 2] * xr[:, 2])[:, None]
        q2 = ((nxt[0] * nxt[0] + nxt[1] * nxt[1]) + nxt[2] * nxt[2])[None, :]
        dot = jax.lax.dot_general(kt, nxt, (((0,), (0,)), ((), ())),
                                  preferred_element_type=jnp.float32)
        o_ref[0, :, bi * npoint:(bi + 1) * npoint] = (q2 + k2) - 2.0 * dot


def _d2t_pallas(xyz, xyzT, nxT, npoint):
    bsz, nn, _ = xyz.shape
    rtot = nxT.shape[1]
    if npoint < 128:
        return pl.pallas_call(
            functools.partial(_d2t_small_body, npoint=npoint),
            out_shape=jax.ShapeDtypeStruct((rtot // 128, nn, 128), jnp.float32),
        )(xyz, xyzT, nxT)
    qq = 128
    nbq = npoint // qq
    return pl.pallas_call(
        _d2_body,
        grid=(bsz, nbq),
        in_specs=[pl.BlockSpec((1, nn, 3), lambda bi, i: (bi, 0, 0)),
                  pl.BlockSpec((1, 3, nn), lambda bi, i: (bi, 0, 0)),
                  pl.BlockSpec((3, qq), lambda bi, i: (0, bi * nbq + i))],
        out_specs=pl.BlockSpec((1, nn, qq), lambda bi, i: (bi * nbq + i, 0, 0)),
        out_shape=jax.ShapeDtypeStruct((rtot // 128, nn, 128), jnp.float32),
    )(xyz, xyzT, nxT)


def _ballq_sc(d2t, nn, npoint, r1, r2, ns1, ns2):
    rtot = d2t.shape[0] * 128
    nw = 32
    total_groups = rtot // 128
    gpw = max(1, total_groups // nw)
    csz = min(nn, 512)
    nchunks = nn // csz
    r1sq = jnp.float32(r1 * r1)
    r2sq = jnp.float32(r2 * r2)
    mesh = plsc.VectorSubcoreMesh(core_axis_name="c", subcore_axis_name="s")

    @functools.partial(
        pl.kernel, mesh=mesh,
        out_type=(jax.ShapeDtypeStruct((rtot * ns1,), jnp.int32),
                  jax.ShapeDtypeStruct((rtot * ns2,), jnp.int32)),
        scratch_types=[pltpu.VMEM((csz, 128), jnp.float32),
                       pltpu.VMEM((128 * ns1,), jnp.int32),
                       pltpu.VMEM((128 * ns2,), jnp.int32)],
    )
    def k(d2_hbm, o1_hbm, o2_hbm, slab, ob1, ob2):
        wid = jax.lax.axis_index("s") * 2 + jax.lax.axis_index("c")
        lane = jax.lax.iota(jnp.int32, 16)
        zero16 = jnp.zeros((16,), jnp.int32)
        for g in range(gpw):
            gq = wid * gpw + g
            if total_groups < nw:
                gq = jnp.minimum(gq, total_groups - 1)

            def cbody(cchunk, cnts, gq=gq):
                pltpu.sync_copy(
                    d2_hbm.at[gq, pl.ds(cchunk * csz, csz)], slab)
                new = []
                for sg in range(8):
                    c1v, c2v = cnts[2 * sg], cnts[2 * sg + 1]
                    qb1 = (sg * 16 + lane) * ns1
                    qb2 = (sg * 16 + lane) * ns2
                    basev = (((gq * 128 + sg * 16 + lane) // npoint) * nn
                             + cchunk * csz)

                    def ibody(i, st, sg=sg, qb1=qb1, qb2=qb2, basev=basev):
                        c1v, c2v = st
                        dv = jnp.squeeze(slab[pl.ds(i, 1), pl.ds(sg * 16, 16)], axis=0)
                        ival = basev + i
                        m2 = dv < r2sq
                        wm2 = m2 & (c2v < ns2)
                        plsc.store_scatter(ob2, [qb2 + c2v], ival, mask=wm2)
                        c2v = c2v + jnp.where(m2, jnp.int32(1), jnp.int32(0))
                        m1 = dv < r1sq
                        wm1 = m1 & (c1v < ns1)
                        plsc.store_scatter(ob1, [qb1 + c1v], ival, mask=wm1)
                        c1v = c1v + jnp.where(m1, jnp.int32(1), jnp.int32(0))
                        return (c1v, c2v)

                    c1v, c2v = jax.lax.fori_loop(
                        0, csz, ibody, (c1v, c2v))
                    new += [c1v, c2v]
                return tuple(new)

            cnts = jax.lax.fori_loop(0, nchunks, cbody,
                                     tuple([zero16] * 16))
            for sg in range(8):
                c1v, c2v = cnts[2 * sg], cnts[2 * sg + 1]
                basev = ((gq * 128 + sg * 16 + lane) // npoint) * nn
                for ob, ns, cv in ((ob1, ns1, c1v), (ob2, ns2, c2v)):
                    qb = (sg * 16 + lane) * ns
                    firsts = plsc.load_gather(ob, [qb])
                    fillv = jnp.where(cv > 0, firsts, basev)

                    def pbody(j, _, ob=ob, qb=qb, fillv=fillv, cv=cv):
                        jv = jnp.full((16,), j, jnp.int32)
                        plsc.store_scatter(ob, [qb + j], fillv,
                                           mask=jv >= cv)
                        return 0

                    jax.lax.fori_loop(0, ns, pbody, 0)
            pltpu.sync_copy(ob1, o1_hbm.at[pl.ds(gq * 128 * ns1, 128 * ns1)])
            pltpu.sync_copy(ob2, o2_hbm.at[pl.ds(gq * 128 * ns2, 128 * ns2)])

    o1, o2 = k(d2t)
    return o1.reshape(rtot, ns1), o2.reshape(rtot, ns2)


def _gather_sc(table, idx):
    v, dpad = table.shape
    m = idx.shape[0]
    nw = 32
    mper = m // nw
    ch = min(128, mper)
    mesh = plsc.VectorSubcoreMesh(core_axis_name="c", subcore_axis_name="s")

    @functools.partial(
        pl.kernel, mesh=mesh,
        out_type=jax.ShapeDtypeStruct((m, dpad), jnp.float32),
        scratch_types=[pltpu.VMEM((ch,), jnp.int32),
                       pltpu.VMEM((ch, dpad), jnp.float32),
                       pltpu.SemaphoreType.DMA],
    )
    def k(t_hbm, i_hbm, o_hbm, idx_v, rows_v, sem):
        wid = jax.lax.axis_index("s") * 2 + jax.lax.axis_index("c")
        base = wid * mper

        def body(j, _):
            off = base + j * ch
            pltpu.sync_copy(i_hbm.at[pl.ds(off, ch)], idx_v)
            pltpu.async_copy(t_hbm.at[idx_v], rows_v, sem).wait()
            pltpu.sync_copy(rows_v, o_hbm.at[pl.ds(off, ch)])
            return 0

        jax.lax.fori_loop(0, mper // ch, body, 0)

    return k(table, idx)


def _fps_body(xs_ref, ys_ref, zs_ref, idx_ref, nx_ref, d_ref, *, npoint):
    bsz, nb, _ = xs_ref.shape
    xs, ys, zs = xs_ref[...], ys_ref[...], zs_ref[...]
    gi = (jax.lax.broadcasted_iota(jnp.int32, (bsz, nb, 128), 1) * 128
          + jax.lax.broadcasted_iota(jnp.int32, (bsz, nb, 128), 2))
    d_ref[...] = jnp.full((bsz, nb, 128), 1e10, jnp.float32)
    idx_ref[...] = jnp.zeros(idx_ref.shape, jnp.int32)
    iota_np = jax.lax.broadcasted_iota(jnp.int32, idx_ref.shape, 1)
    iota_nx = jax.lax.broadcasted_iota(jnp.int32, nx_ref.shape, 2)

    def coords_of(li):
        sel = gi == li
        lx = jnp.sum(jnp.where(sel, xs, 0.0), axis=(1, 2), keepdims=True)
        ly = jnp.sum(jnp.where(sel, ys, 0.0), axis=(1, 2), keepdims=True)
        lz = jnp.sum(jnp.where(sel, zs, 0.0), axis=(1, 2), keepdims=True)
        return lx, ly, lz

    def body(i, last):
        lx, ly, lz = coords_of(last)
        dx, dy, dz = xs - lx, ys - ly, zs - lz
        d = (dx * dx + dy * dy) + dz * dz
        nd = jnp.minimum(d_ref[...], d)
        d_ref[...] = nd
        m = jnp.max(nd, axis=(1, 2), keepdims=True)
        nxt = jnp.min(jnp.where(nd == m, gi, jnp.int32(2 ** 30)),
                      axis=(1, 2), keepdims=True)
        idx_ref[...] = jnp.where(iota_np == i, nxt[:, :, 0], idx_ref[...])
        lc = jnp.concatenate([lx, ly, lz], axis=1)
        nx_ref[...] = jnp.where(iota_nx == i - 1, lc, nx_ref[...])
        return nxt

    last = jax.lax.fori_loop(1, npoint, body,
                             jnp.zeros((bsz, 1, 1), jnp.int32))
    lx, ly, lz = coords_of(last)
    lc = jnp.concatenate([lx, ly, lz], axis=1)
    nx_ref[...] = jnp.where(iota_nx == npoint - 1, lc, nx_ref[...])


def _fps_pallas(xyz, npoint):
    bsz, n, _ = xyz.shape
    nb = n // 128
    xs = xyz[:, :, 0].reshape(bsz, nb, 128)
    ys = xyz[:, :, 1].reshape(bsz, nb, 128)
    zs = xyz[:, :, 2].reshape(bsz, nb, 128)
    idx, nxyz = pl.pallas_call(
        functools.partial(_fps_body, npoint=npoint),
        out_shape=(jax.ShapeDtypeStruct((bsz, npoint), jnp.int32),
                   jax.ShapeDtypeStruct((bsz, 3, npoint), jnp.float32)),
        scratch_shapes=[pltpu.VMEM((bsz, nb, 128), jnp.float32)],
    )(xs, ys, zs)
    return idx, jnp.transpose(nxyz, (0, 2, 1))


def _fps_jax(xyz, npoint):
    bn, nn, _ = xyz.shape
    dists = jnp.full((bn, nn), 1e10, dtype=jnp.float32)
    idxs = jnp.zeros((bn, npoint), dtype=jnp.int32)

    def body(i, state):
        dists, idxs = state
        last = idxs[:, i - 1]
        last_pt = jnp.take_along_axis(xyz, last[:, None, None], axis=1)
        d = jnp.sum((xyz - last_pt) ** 2, axis=-1)
        dists = jnp.minimum(dists, d)
        nxt = jnp.argmax(dists, axis=1).astype(jnp.int32)
        idxs = idxs.at[:, i].set(nxt)
        return (dists, idxs)

    dists, idxs = jax.lax.fori_loop(1, npoint, body, (dists, idxs))
    return idxs


def _sqdist(a, b):
    a2 = jnp.sum(a * a, axis=-1)
    b2 = jnp.sum(b * b, axis=-1)
    return a2[:, :, None] + b2[:, None, :] - 2.0 * jnp.einsum('bnd,bmd->bnm', a, b)


def _ball_query_jax(radius, nsample, xyz, new_xyz):
    nn = xyz.shape[1]
    d2 = _sqdist(new_xyz, xyz)
    mask = d2 < radius * radius
    ar = jnp.arange(nn, dtype=jnp.int32)
    keys = jnp.where(mask, -ar[None, None, :], jnp.int32(-(nn + 1)))
    vals, _ = jax.lax.top_k(keys, nsample)
    idx = -vals
    first = jnp.where(idx[:, :, :1] >= nn, 0, idx[:, :, :1])
    idx = jnp.where(idx >= nn, first, idx)
    return idx


def _sa_level(xyz, featrows, k, sa_k):
    bsz, n, _ = xyz.shape
    npoint = _NPOINTS[k]
    fid, new_xyz = _fps_pallas(xyz, npoint)
    nxyz_rows = new_xyz.reshape(bsz * npoint, 3)
    table = jnp.concatenate([xyz, featrows], axis=-1)
    d = table.shape[-1]
    dpad = _rup(d, 128)
    if dpad != d:
        table = jnp.concatenate(
            [table, jnp.zeros((bsz, n, dpad - d), jnp.float32)], axis=-1)
    (r1, r2), (ns1, ns2) = _RADIUS[k], _NSAMPLE[k]
    boff = (jnp.arange(bsz, dtype=jnp.int32) * n)[:, None, None]
    idx1 = (_ball_query_jax(r1, ns1, xyz, new_xyz) + boff).reshape(bsz * npoint, ns1)
    idx2 = (_ball_query_jax(r2, ns2, xyz, new_xyz) + boff).reshape(bsz * npoint, ns2)
    table2d = table.reshape(bsz * n, dpad)
    outs = []
    for s, (idxs, ns) in enumerate(((idx1, ns1), (idx2, ns2))):
        g = _gather_sc(table2d, idxs.reshape(-1))
        o = _sa_mlp(g, nxyz_rows, sa_k[s], ns)
        outs.append(o.reshape(bsz, npoint, -1))
    return new_xyz, jnp.concatenate(outs, axis=-1)


def kernel(pointcloud, sa_params, fp_params):
    xyz = pointcloud[:, :, 0:3]
    featrows = pointcloud[:, :, 3:]
    l_xyz = [xyz]
    l_feat = [featrows]
    for k in range(4):
        nx, nf = _sa_level(l_xyz[k], l_feat[k], k, sa_params[k])
        l_xyz.append(nx)
        l_feat.append(nf)
    for i in range(-1, -5, -1):
        l_feat[i - 1] = _fp(l_xyz[i - 1], l_xyz[i], l_feat[i - 1], l_feat[i],
                            fp_params[i])
    return l_feat[0]
